# baseline (speedup 1.0000x reference)
import functools

import jax
import jax.numpy as jnp
from jax import lax
from jax.experimental import pallas as pl
from jax.experimental.pallas import tpu as pltpu
from jax.experimental.pallas import tpu_sc as plsc

B, N, D = 8, 4096, 64
G, K = 512, 24
KP = 32
F = 16
GT = 64

_SC_NC, _SC_NS = 2, 16
_SC_NW = _SC_NC * _SC_NS


def _fps_body(xyz_ref, idx_ref, cx_ref, cy_ref, cz_ref):
    xb = xyz_ref[0]
    yb = xyz_ref[1]
    zb = xyz_ref[2]
    iota = lax.broadcasted_iota(jnp.int32, (B, N), 1)

    def body(i, carry):
        dist, far = carry
        eq = iota == far
        cx = jnp.sum(jnp.where(eq, xb, 0.0), axis=1, keepdims=True)
        cy = jnp.sum(jnp.where(eq, yb, 0.0), axis=1, keepdims=True)
        cz = jnp.sum(jnp.where(eq, zb, 0.0), axis=1, keepdims=True)
        d = (xb - cx) ** 2 + (yb - cy) ** 2 + (zb - cz) ** 2
        dist = jnp.minimum(dist, d)
        m = jnp.max(dist, axis=1, keepdims=True)
        far2 = jnp.min(jnp.where(dist == m, iota, N), axis=1, keepdims=True)
        idx_ref[pl.ds(i, 1), :, :] = far[None]
        cx_ref[pl.ds(i, 1), :, :] = cx[None]
        cy_ref[pl.ds(i, 1), :, :] = cy[None]
        cz_ref[pl.ds(i, 1), :, :] = cz[None]
        return dist, far2

    dist0 = jnp.full((B, N), 1e10, jnp.float32)
    far0 = jnp.zeros((B, 1), jnp.int32)
    lax.fori_loop(0, G, body, (dist0, far0))


def _run_fps(xyz_t):
    f32 = jnp.float32
    return pl.pallas_call(
        _fps_body,
        out_shape=[
            jax.ShapeDtypeStruct((G, B, 1), jnp.int32),
            jax.ShapeDtypeStruct((G, B, 1), f32),
            jax.ShapeDtypeStruct((G, B, 1), f32),
            jax.ShapeDtypeStruct((G, B, 1), f32),
        ],
    )(xyz_t)


CH = 32


def _ce(v, ix, i, j):
    p = v[i] <= v[j]
    vi = jnp.where(p, v[i], v[j])
    vj = jnp.where(p, v[j], v[i])
    ii = jnp.where(p, ix[i], ix[j])
    ij = jnp.where(p, ix[j], ix[i])
    v[i], v[j], ix[i], ix[j] = vi, vj, ii, ij


def _bitonic_sort(v, ix):
    n = len(v)
    k = 2
    while k <= n:
        j = k >> 1
        while j >= 1:
            for i in range(n):
                l = i ^ j
                if l > i:
                    if (i & k) == 0:
                        _ce(v, ix, i, l)
                    else:
                        _ce(v, ix, l, i)
            j >>= 1
        k <<= 1


def _knn_body(xc_ref, q_ref, base_ref, o_ref):
    qx = q_ref[0, 0]
    qy = q_ref[0, 1]
    qz = q_ref[0, 2]
    base = base_ref[0, 0]

    inf = jnp.full((8, 128), jnp.inf, jnp.float32)
    zero = jnp.zeros((8, 128), jnp.int32)
    init = tuple([inf] * CH + [zero] * CH)

    basev = jnp.broadcast_to(base, (8, 128))

    def block_step(m, carry):
        run = list(carry)
        Xm = xc_ref[0, 0, pl.ds(m, 1), :, :].reshape(8, 128)
        Ym = xc_ref[0, 1, pl.ds(m, 1), :, :].reshape(8, 128)
        Zm = xc_ref[0, 2, pl.ds(m, 1), :, :].reshape(8, 128)
        for cj in range(128 // CH):
            rv = run[:CH]
            ri = run[CH:]
            cv, ci = [], []
            for j in range(CH):
                lj = cj * CH + j
                xj = Xm[:, lj:lj + 1]
                yj = Ym[:, lj:lj + 1]
                zj = Zm[:, lj:lj + 1]
                d = (qx - xj) ** 2 + (qy - yj) ** 2 + (qz - zj) ** 2
                cv.append(d)
                ci.append(basev + (m * 128 + lj))
            _bitonic_sort(cv, ci)
            mv, mi = [], []
            for i in range(CH):
                p = rv[i] <= cv[CH - 1 - i]
                mv.append(jnp.where(p, rv[i], cv[CH - 1 - i]))
                mi.append(jnp.where(p, ri[i], ci[CH - 1 - i]))
            for stride in (16, 8, 4, 2, 1):
                for i in range(CH):
                    if (i % (2 * stride)) < stride:
                        _ce(mv, mi, i, i + stride)
            run = mv + mi
        return tuple(run)

    fin = lax.fori_loop(0, N // 128, block_step, init)
    for k in range(K):
        o_ref[0, k] = fin[CH + k]


def _run_knn(xc, q, basearr):
    return pl.pallas_call(
        _knn_body,
        grid=(B // 2,),
        in_specs=[
            pl.BlockSpec((1, 3, N // 128, 8, 128), lambda p: (p, 0, 0, 0, 0)),
            pl.BlockSpec((1, 3, 8, 128), lambda p: (p, 0, 0, 0)),
            pl.BlockSpec((1, 1, 8, 1), lambda p: (p, 0, 0, 0)),
        ],
        out_specs=pl.BlockSpec((1, K, 8, 128), lambda p: (p, 0, 0, 0)),
        out_shape=jax.ShapeDtypeStruct((B // 2, K, 8, 128), jnp.int32),
        compiler_params=pltpu.CompilerParams(
            dimension_semantics=("parallel",)),
    )(xc, q, basearr)


def _sc_gather(table, idx, dcols, chunk):
    rows = idx.shape[0]
    rows_per_w = rows // _SC_NW
    nchunks = rows_per_w // chunk
    mesh = plsc.VectorSubcoreMesh(core_axis_name="c", subcore_axis_name="s")

    @functools.partial(
        pl.kernel,
        mesh=mesh,
        out_type=jax.ShapeDtypeStruct((rows, dcols), jnp.float32),
        scratch_types=[
            pltpu.VMEM((chunk,), jnp.int32),
            pltpu.VMEM((chunk, dcols), jnp.float32),
            pltpu.SemaphoreType.DMA,
        ],
    )
    def gather_kernel(table_hbm, idx_hbm, out_hbm, idx_v, rows_v, sem):
        wid = lax.axis_index("s") * _SC_NC + lax.axis_index("c")
        base = wid * rows_per_w

        @pl.loop(0, nchunks)
        def _(c):
            off = base + c * chunk
            pltpu.sync_copy(idx_hbm.at[pl.ds(off, chunk)], idx_v)
            pltpu.async_copy(table_hbm.at[idx_v], rows_v, sem).wait()
            pltpu.sync_copy(rows_v, out_hbm.at[pl.ds(off, chunk)])

    return gather_kernel(table, idx)


GT2 = 32
TW = 2 * D


def _fin_body(g_ref, a_ref, gp_ref, aug_ref):
    g = g_ref[...]
    a = a_ref[...]
    gp_ref[...] = jnp.concatenate(
        [g[..., 0:D], jnp.broadcast_to(a[:, :, 0:D], (GT2, K, D))], axis=-1)
    gx = g[..., D:D + F]
    ax = a[..., D:D + F]
    diff = gx - ax
    sq = diff * diff
    ad = jnp.sqrt(sq[..., 0:1] + sq[..., 1:2] + sq[..., 2:3])
    ab = jnp.broadcast_to(ax[:, :, 0:3], (GT2, K, 3))
    aug = jnp.concatenate(
        [ad, diff[..., 0:3], ab, gx[..., 0:3],
         jnp.zeros((GT2, K, F - 10), jnp.float32)], axis=-1)
    mean = jnp.mean(aug, axis=1, keepdims=True)
    c = aug - mean
    std = jnp.sqrt(jnp.sum(c * c, axis=1, keepdims=True) * (1.0 / (K - 1)))
    aug_ref[...] = c / (std + 1e-8)


def _run_fin(g_knn, g_anchor):
    return pl.pallas_call(
        _fin_body,
        grid=(B * G // GT2,),
        in_specs=[
            pl.BlockSpec((GT2, K, TW), lambda i: (i, 0, 0)),
            pl.BlockSpec((GT2, 1, TW), lambda i: (i, 0, 0)),
        ],
        out_specs=[
            pl.BlockSpec((GT2, K, TW), lambda i: (i, 0, 0)),
            pl.BlockSpec((GT2, K, F), lambda i: (i, 0, 0)),
        ],
        out_shape=[
            jax.ShapeDtypeStruct((B * G, K, TW), jnp.float32),
            jax.ShapeDtypeStruct((B * G, K, F), jnp.float32),
        ],
        compiler_params=pltpu.CompilerParams(
            dimension_semantics=("parallel",)),
    )(g_knn, g_anchor)


def kernel(xyz, points):
    xyz_t = jnp.transpose(xyz, (2, 0, 1))

    fps_idx, cx, cy, cz = _run_fps(xyz_t)

    new_xyz = jnp.transpose(jnp.concatenate([cx, cy, cz], axis=-1), (1, 0, 2))

    xc = jnp.broadcast_to(
        xyz_t.reshape(3, B // 2, 2, N // 128, 128)[:, :, :, None, :, :],
        (3, B // 2, 2, 4, N // 128, 128))
    xc = jnp.transpose(xc, (1, 0, 4, 2, 3, 5)).reshape(B // 2, 3, N // 128, 8, 128)
    nc = jnp.stack([cx[..., 0], cy[..., 0], cz[..., 0]], axis=0)
    nc5 = nc.reshape(3, 4, 128, B // 2, 2)
    qq = jnp.transpose(nc5, (3, 0, 4, 1, 2)).reshape(B // 2, 3, 8, 128)
    basearr = jnp.broadcast_to(
        jnp.arange(B, dtype=jnp.int32).reshape(B // 2, 2, 1) * N,
        (B // 2, 2, 4)).reshape(B // 2, 1, 8, 1)
    knn = _run_knn(xc, qq, basearr)

    knn_flat = (jnp.transpose(knn.reshape(B // 2, K, 2, 4, 128),
                              (0, 2, 3, 4, 1)).reshape(-1))
    fps_flat = (jnp.transpose(fps_idx[..., 0])
                + jnp.arange(B, dtype=jnp.int32)[:, None] * N).reshape(-1)

    table = jnp.concatenate(
        [points.reshape(B * N, D), xyz.reshape(B * N, 3),
         jnp.zeros((B * N, TW - D - 3), jnp.float32)], axis=-1)

    g_knn = _sc_gather(table, knn_flat, TW, chunk=768)
    g_anchor = _sc_gather(table, fps_flat, TW, chunk=B * G // _SC_NW)

    gp, aug = _run_fin(g_knn.reshape(B * G, K, TW),
                       g_anchor.reshape(B * G, 1, TW))

    augmented = aug[..., :10].reshape(B, G, K, 10)
    grouped_points = gp.reshape(B, G, K, 2 * D)
    return new_xyz, augmented, grouped_points


# speedup vs baseline: 1.0533x; 1.0533x over previous
import functools

import jax
import jax.numpy as jnp
from jax import lax
from jax.experimental import pallas as pl
from jax.experimental.pallas import tpu as pltpu
from jax.experimental.pallas import tpu_sc as plsc

B, N, D = 8, 4096, 64
G, K = 512, 24
KP = 32
F = 16
GT = 64

_SC_NC, _SC_NS = 2, 16
_SC_NW = _SC_NC * _SC_NS


def _fps_body(xyz_ref, idx_ref, cx_ref, cy_ref, cz_ref):
    xb = xyz_ref[0]
    yb = xyz_ref[1]
    zb = xyz_ref[2]
    iota = lax.broadcasted_iota(jnp.int32, (B, N), 1)

    def body(i, carry):
        dist, far = carry
        eq = iota == far
        cx = jnp.sum(jnp.where(eq, xb, 0.0), axis=1, keepdims=True)
        cy = jnp.sum(jnp.where(eq, yb, 0.0), axis=1, keepdims=True)
        cz = jnp.sum(jnp.where(eq, zb, 0.0), axis=1, keepdims=True)
        d = (xb - cx) ** 2 + (yb - cy) ** 2 + (zb - cz) ** 2
        dist = jnp.minimum(dist, d)
        m = jnp.max(dist, axis=1, keepdims=True)
        far2 = jnp.min(jnp.where(dist == m, iota, N), axis=1, keepdims=True)
        idx_ref[pl.ds(i, 1), :, :] = far[None]
        cx_ref[pl.ds(i, 1), :, :] = cx[None]
        cy_ref[pl.ds(i, 1), :, :] = cy[None]
        cz_ref[pl.ds(i, 1), :, :] = cz[None]
        return dist, far2

    dist0 = jnp.full((B, N), 1e10, jnp.float32)
    far0 = jnp.zeros((B, 1), jnp.int32)
    lax.fori_loop(0, G, body, (dist0, far0))


def _run_fps(xyz_t):
    f32 = jnp.float32
    return pl.pallas_call(
        _fps_body,
        out_shape=[
            jax.ShapeDtypeStruct((G, B, 1), jnp.int32),
            jax.ShapeDtypeStruct((G, B, 1), f32),
            jax.ShapeDtypeStruct((G, B, 1), f32),
            jax.ShapeDtypeStruct((G, B, 1), f32),
        ],
    )(xyz_t)


CH = 32


def _ce(v, ix, i, j):
    p = v[i] <= v[j]
    vi = jnp.where(p, v[i], v[j])
    vj = jnp.where(p, v[j], v[i])
    ii = jnp.where(p, ix[i], ix[j])
    ij = jnp.where(p, ix[j], ix[i])
    v[i], v[j], ix[i], ix[j] = vi, vj, ii, ij


def _bitonic_sort(v, ix):
    n = len(v)
    k = 2
    while k <= n:
        j = k >> 1
        while j >= 1:
            for i in range(n):
                l = i ^ j
                if l > i:
                    if (i & k) == 0:
                        _ce(v, ix, i, l)
                    else:
                        _ce(v, ix, l, i)
            j >>= 1
        k <<= 1


def _knn_body(xc_ref, q_ref, base_ref, o_ref):
    qx = q_ref[0, 0]
    qy = q_ref[0, 1]
    qz = q_ref[0, 2]
    base = base_ref[0, 0]

    inf = jnp.full((8, 128), jnp.inf, jnp.float32)
    zero = jnp.zeros((8, 128), jnp.int32)
    init = tuple([inf] * CH + [zero] * CH)

    basev = jnp.broadcast_to(base, (8, 128))

    def block_step(m, carry):
        run = list(carry)
        Xm = xc_ref[0, 0, pl.ds(m, 1), :, :].reshape(8, 128)
        Ym = xc_ref[0, 1, pl.ds(m, 1), :, :].reshape(8, 128)
        Zm = xc_ref[0, 2, pl.ds(m, 1), :, :].reshape(8, 128)
        for cj in range(128 // CH):
            rv = run[:CH]
            ri = run[CH:]
            cv, ci = [], []
            for j in range(CH):
                lj = cj * CH + j
                xj = Xm[:, lj:lj + 1]
                yj = Ym[:, lj:lj + 1]
                zj = Zm[:, lj:lj + 1]
                d = (qx - xj) ** 2 + (qy - yj) ** 2 + (qz - zj) ** 2
                cv.append(d)
                ci.append(basev + (m * 128 + lj))
            _bitonic_sort(cv, ci)
            mv, mi = [], []
            for i in range(CH):
                p = rv[i] <= cv[CH - 1 - i]
                mv.append(jnp.where(p, rv[i], cv[CH - 1 - i]))
                mi.append(jnp.where(p, ri[i], ci[CH - 1 - i]))
            for stride in (16, 8, 4, 2, 1):
                for i in range(CH):
                    if (i % (2 * stride)) < stride:
                        _ce(mv, mi, i, i + stride)
            run = mv + mi
        return tuple(run)

    fin = lax.fori_loop(0, N // 128, block_step, init)
    for k in range(K):
        o_ref[0, k] = fin[CH + k]


def _run_knn(xc, q, basearr):
    return pl.pallas_call(
        _knn_body,
        grid=(B // 2,),
        in_specs=[
            pl.BlockSpec((1, 3, N // 128, 8, 128), lambda p: (p, 0, 0, 0, 0)),
            pl.BlockSpec((1, 3, 8, 128), lambda p: (p, 0, 0, 0)),
            pl.BlockSpec((1, 1, 8, 1), lambda p: (p, 0, 0, 0)),
        ],
        out_specs=pl.BlockSpec((1, K, 8, 128), lambda p: (p, 0, 0, 0)),
        out_shape=jax.ShapeDtypeStruct((B // 2, K, 8, 128), jnp.int32),
        compiler_params=pltpu.CompilerParams(
            dimension_semantics=("parallel",)),
    )(xc, q, basearr)


def _sc_gather(table, idx, dcols, chunk):
    rows = idx.shape[0]
    rows_per_w = rows // _SC_NW
    nchunks = rows_per_w // chunk
    mesh = plsc.VectorSubcoreMesh(core_axis_name="c", subcore_axis_name="s")

    @functools.partial(
        pl.kernel,
        mesh=mesh,
        out_type=jax.ShapeDtypeStruct((rows, dcols), jnp.float32),
        scratch_types=[
            pltpu.VMEM((chunk,), jnp.int32),
            pltpu.VMEM((chunk, dcols), jnp.float32),
            pltpu.SemaphoreType.DMA,
        ],
    )
    def gather_kernel(table_hbm, idx_hbm, out_hbm, idx_v, rows_v, sem):
        wid = lax.axis_index("s") * _SC_NC + lax.axis_index("c")
        base = wid * rows_per_w

        @pl.loop(0, nchunks)
        def _(c):
            off = base + c * chunk
            pltpu.sync_copy(idx_hbm.at[pl.ds(off, chunk)], idx_v)
            pltpu.async_copy(table_hbm.at[idx_v], rows_v, sem).wait()
            pltpu.sync_copy(rows_v, out_hbm.at[pl.ds(off, chunk)])

    return gather_kernel(table, idx)


GT2 = 64
TW = 2 * D


def _fin_body(g_ref, a_ref, gp_ref, aug_ref):
    g = g_ref[...]
    a = a_ref[...]
    gp_ref[..., 0:D] = g[..., 0:D]
    gp_ref[..., D:TW] = jnp.broadcast_to(a[:, :, 0:D], (GT2, K, D))
    gx = g[..., D:D + F]
    ax = a[..., D:D + F]
    diff = gx - ax
    sq = diff * diff
    ad = jnp.sqrt(sq[..., 0:1] + sq[..., 1:2] + sq[..., 2:3])
    ab = jnp.broadcast_to(ax[:, :, 0:3], (GT2, K, 3))
    aug = jnp.concatenate(
        [ad, diff[..., 0:3], ab, gx[..., 0:3],
         jnp.zeros((GT2, K, F - 10), jnp.float32)], axis=-1)
    mean = jnp.mean(aug, axis=1, keepdims=True)
    c = aug - mean
    std = jnp.sqrt(jnp.sum(c * c, axis=1, keepdims=True) * (1.0 / (K - 1)))
    aug_ref[...] = c / (std + 1e-8)


def _run_fin(g_knn, g_anchor):
    return pl.pallas_call(
        _fin_body,
        grid=(B * G // GT2,),
        in_specs=[
            pl.BlockSpec((GT2, K, TW), lambda i: (i, 0, 0)),
            pl.BlockSpec((GT2, 1, TW), lambda i: (i, 0, 0)),
        ],
        out_specs=[
            pl.BlockSpec((GT2, K, TW), lambda i: (i, 0, 0)),
            pl.BlockSpec((GT2, K, F), lambda i: (i, 0, 0)),
        ],
        out_shape=[
            jax.ShapeDtypeStruct((B * G, K, TW), jnp.float32),
            jax.ShapeDtypeStruct((B * G, K, F), jnp.float32),
        ],
        compiler_params=pltpu.CompilerParams(
            dimension_semantics=("parallel",)),
    )(g_knn, g_anchor)


def kernel(xyz, points):
    xyz_t = jnp.transpose(xyz, (2, 0, 1))

    fps_idx, cx, cy, cz = _run_fps(xyz_t)

    new_xyz = jnp.transpose(jnp.concatenate([cx, cy, cz], axis=-1), (1, 0, 2))

    xc = jnp.broadcast_to(
        xyz_t.reshape(3, B // 2, 2, N // 128, 128)[:, :, :, None, :, :],
        (3, B // 2, 2, 4, N // 128, 128))
    xc = jnp.transpose(xc, (1, 0, 4, 2, 3, 5)).reshape(B // 2, 3, N // 128, 8, 128)
    nc = jnp.stack([cx[..., 0], cy[..., 0], cz[..., 0]], axis=0)
    nc5 = nc.reshape(3, 4, 128, B // 2, 2)
    qq = jnp.transpose(nc5, (3, 0, 4, 1, 2)).reshape(B // 2, 3, 8, 128)
    basearr = jnp.broadcast_to(
        jnp.arange(B, dtype=jnp.int32).reshape(B // 2, 2, 1) * N,
        (B // 2, 2, 4)).reshape(B // 2, 1, 8, 1)
    knn = _run_knn(xc, qq, basearr)

    knn_flat = (jnp.transpose(knn.reshape(B // 2, K, 2, 4, 128),
                              (0, 2, 3, 4, 1)).reshape(-1))
    fps_flat = (jnp.transpose(fps_idx[..., 0])
                + jnp.arange(B, dtype=jnp.int32)[:, None] * N).reshape(-1)

    table = jnp.concatenate(
        [points.reshape(B * N, D), xyz.reshape(B * N, 3),
         jnp.zeros((B * N, TW - D - 3), jnp.float32)], axis=-1)

    g_knn = _sc_gather(table, knn_flat, TW, chunk=768)
    g_anchor = _sc_gather(table, fps_flat, TW, chunk=B * G // _SC_NW)

    gp, aug = _run_fin(g_knn.reshape(B * G, K, TW),
                       g_anchor.reshape(B * G, 1, TW))

    augmented = aug[..., :10].reshape(B, G, K, 10)
    grouped_points = gp.reshape(B, G, K, 2 * D)
    return new_xyz, augmented, grouped_points


# speedup vs baseline: 1.1014x; 1.0457x over previous
import functools

import jax
import jax.numpy as jnp
from jax import lax
from jax.experimental import pallas as pl
from jax.experimental.pallas import tpu as pltpu
from jax.experimental.pallas import tpu_sc as plsc

B, N, D = 8, 4096, 64
G, K = 512, 24
KP = 32
F = 16
GT = 64

_SC_NC, _SC_NS = 2, 16
_SC_NW = _SC_NC * _SC_NS


def _fps_body(xyz_ref, idx_ref, cx_ref, cy_ref, cz_ref):
    xb = xyz_ref[0]
    yb = xyz_ref[1]
    zb = xyz_ref[2]
    iota = lax.broadcasted_iota(jnp.int32, (B, N), 1)

    def fold(v, op):
        parts = [v[:, c * 128:(c + 1) * 128] for c in range(N // 128)]
        while len(parts) > 1:
            parts = [op(parts[i], parts[i + 1]) if i + 1 < len(parts)
                     else parts[i] for i in range(0, len(parts), 2)]
        return parts[0]

    def body(i, carry):
        dist, far = carry
        eq = iota == far
        cx = jnp.sum(fold(jnp.where(eq, xb, 0.0), jnp.add),
                     axis=1, keepdims=True)
        cy = jnp.sum(fold(jnp.where(eq, yb, 0.0), jnp.add),
                     axis=1, keepdims=True)
        cz = jnp.sum(fold(jnp.where(eq, zb, 0.0), jnp.add),
                     axis=1, keepdims=True)
        d = (xb - cx) ** 2 + (yb - cy) ** 2 + (zb - cz) ** 2
        dist = jnp.minimum(dist, d)
        m = jnp.max(fold(dist, jnp.maximum), axis=1, keepdims=True)
        far2 = jnp.min(fold(jnp.where(dist == m, iota, N), jnp.minimum),
                       axis=1, keepdims=True)
        idx_ref[pl.ds(i, 1), :, :] = far[None]
        cx_ref[pl.ds(i, 1), :, :] = cx[None]
        cy_ref[pl.ds(i, 1), :, :] = cy[None]
        cz_ref[pl.ds(i, 1), :, :] = cz[None]
        return dist, far2

    dist0 = jnp.full((B, N), 1e10, jnp.float32)
    far0 = jnp.zeros((B, 1), jnp.int32)
    lax.fori_loop(0, G, body, (dist0, far0))


def _run_fps(xyz_t):
    f32 = jnp.float32
    return pl.pallas_call(
        _fps_body,
        out_shape=[
            jax.ShapeDtypeStruct((G, B, 1), jnp.int32),
            jax.ShapeDtypeStruct((G, B, 1), f32),
            jax.ShapeDtypeStruct((G, B, 1), f32),
            jax.ShapeDtypeStruct((G, B, 1), f32),
        ],
    )(xyz_t)


CH = 32


def _ce(v, ix, i, j):
    p = v[i] <= v[j]
    vi = jnp.where(p, v[i], v[j])
    vj = jnp.where(p, v[j], v[i])
    ii = jnp.where(p, ix[i], ix[j])
    ij = jnp.where(p, ix[j], ix[i])
    v[i], v[j], ix[i], ix[j] = vi, vj, ii, ij


def _oems_pairs(n):
    pairs = []

    def merge(lo, m, r):
        step = r * 2
        if step < m:
            merge(lo, m, step)
            merge(lo + r, m, step)
            for i in range(lo + r, lo + m - r, step):
                pairs.append((i, i + r))
        else:
            pairs.append((lo, lo + r))

    def sort(lo, m):
        if m > 1:
            h = m // 2
            sort(lo, h)
            sort(lo + h, h)
            merge(lo, m, 1)

    sort(0, n)
    return pairs


_SORT_NET = _oems_pairs(CH)


def _bitonic_sort(v, ix):
    for i, j in _SORT_NET:
        _ce(v, ix, i, j)


def _knn_body(xc_ref, q_ref, base_ref, o_ref):
    qx = q_ref[0, 0]
    qy = q_ref[0, 1]
    qz = q_ref[0, 2]
    base = base_ref[0, 0]

    inf = jnp.full((8, 128), jnp.inf, jnp.float32)
    zero = jnp.zeros((8, 128), jnp.int32)
    init = tuple([inf] * CH + [zero] * CH)

    basev = jnp.broadcast_to(base, (8, 128))

    def block_step(m, carry):
        run = list(carry)
        Xm = xc_ref[0, 0, pl.ds(m, 1), :, :].reshape(8, 128)
        Ym = xc_ref[0, 1, pl.ds(m, 1), :, :].reshape(8, 128)
        Zm = xc_ref[0, 2, pl.ds(m, 1), :, :].reshape(8, 128)
        for cj in range(128 // CH):
            rv = run[:CH]
            ri = run[CH:]
            cv, ci = [], []
            for j in range(CH):
                lj = cj * CH + j
                xj = Xm[:, lj:lj + 1]
                yj = Ym[:, lj:lj + 1]
                zj = Zm[:, lj:lj + 1]
                d = (qx - xj) ** 2 + (qy - yj) ** 2 + (qz - zj) ** 2
                cv.append(d)
                ci.append(basev + (m * 128 + lj))
            _bitonic_sort(cv, ci)
            mv, mi = [], []
            for i in range(CH):
                p = rv[i] <= cv[CH - 1 - i]
                mv.append(jnp.where(p, rv[i], cv[CH - 1 - i]))
                mi.append(jnp.where(p, ri[i], ci[CH - 1 - i]))
            for stride in (16, 8, 4, 2, 1):
                for i in range(CH):
                    if (i % (2 * stride)) < stride:
                        _ce(mv, mi, i, i + stride)
            run = mv + mi
        return tuple(run)

    fin = lax.fori_loop(0, N // 128, block_step, init)
    for k in range(K):
        o_ref[0, k] = fin[CH + k]


def _run_knn(xc, q, basearr):
    return pl.pallas_call(
        _knn_body,
        grid=(B // 2,),
        in_specs=[
            pl.BlockSpec((1, 3, N // 128, 8, 128), lambda p: (p, 0, 0, 0, 0)),
            pl.BlockSpec((1, 3, 8, 128), lambda p: (p, 0, 0, 0)),
            pl.BlockSpec((1, 1, 8, 1), lambda p: (p, 0, 0, 0)),
        ],
        out_specs=pl.BlockSpec((1, K, 8, 128), lambda p: (p, 0, 0, 0)),
        out_shape=jax.ShapeDtypeStruct((B // 2, K, 8, 128), jnp.int32),
        compiler_params=pltpu.CompilerParams(
            dimension_semantics=("parallel",)),
    )(xc, q, basearr)


def _sc_gather(table, idx, dcols, chunk):
    rows = idx.shape[0]
    rows_per_w = rows // _SC_NW
    nchunks = rows_per_w // chunk
    mesh = plsc.VectorSubcoreMesh(core_axis_name="c", subcore_axis_name="s")

    @functools.partial(
        pl.kernel,
        mesh=mesh,
        out_type=jax.ShapeDtypeStruct((rows, dcols), jnp.float32),
        scratch_types=[
            pltpu.VMEM((chunk,), jnp.int32),
            pltpu.VMEM((chunk, dcols), jnp.float32),
            pltpu.SemaphoreType.DMA,
        ],
    )
    def gather_kernel(table_hbm, idx_hbm, out_hbm, idx_v, rows_v, sem):
        wid = lax.axis_index("s") * _SC_NC + lax.axis_index("c")
        base = wid * rows_per_w

        @pl.loop(0, nchunks)
        def _(c):
            off = base + c * chunk
            pltpu.sync_copy(idx_hbm.at[pl.ds(off, chunk)], idx_v)
            pltpu.async_copy(table_hbm.at[idx_v], rows_v, sem).wait()
            pltpu.sync_copy(rows_v, out_hbm.at[pl.ds(off, chunk)])

    return gather_kernel(table, idx)


GT2 = 64
TW = 2 * D


def _fin_body(g_ref, a_ref, gp_ref, aug_ref):
    g = g_ref[...]
    a = a_ref[...]
    gp_ref[..., 0:D] = g[..., 0:D]
    gp_ref[..., D:TW] = jnp.broadcast_to(a[:, :, 0:D], (GT2, K, D))
    gx = g[..., D:D + F]
    ax = a[..., D:D + F]
    diff = gx - ax
    sq = diff * diff
    ad = jnp.sqrt(sq[..., 0:1] + sq[..., 1:2] + sq[..., 2:3])
    ab = jnp.broadcast_to(ax[:, :, 0:3], (GT2, K, 3))
    aug = jnp.concatenate(
        [ad, diff[..., 0:3], ab, gx[..., 0:3],
         jnp.zeros((GT2, K, F - 10), jnp.float32)], axis=-1)
    mean = jnp.mean(aug, axis=1, keepdims=True)
    c = aug - mean
    std = jnp.sqrt(jnp.sum(c * c, axis=1, keepdims=True) * (1.0 / (K - 1)))
    aug_ref[...] = c / (std + 1e-8)


def _run_fin(g_knn, g_anchor):
    return pl.pallas_call(
        _fin_body,
        grid=(B * G // GT2,),
        in_specs=[
            pl.BlockSpec((GT2, K, TW), lambda i: (i, 0, 0)),
            pl.BlockSpec((GT2, 1, TW), lambda i: (i, 0, 0)),
        ],
        out_specs=[
            pl.BlockSpec((GT2, K, TW), lambda i: (i, 0, 0)),
            pl.BlockSpec((GT2, K, F), lambda i: (i, 0, 0)),
        ],
        out_shape=[
            jax.ShapeDtypeStruct((B * G, K, TW), jnp.float32),
            jax.ShapeDtypeStruct((B * G, K, F), jnp.float32),
        ],
        compiler_params=pltpu.CompilerParams(
            dimension_semantics=("parallel",)),
    )(g_knn, g_anchor)


def kernel(xyz, points):
    xyz_t = jnp.transpose(xyz, (2, 0, 1))

    fps_idx, cx, cy, cz = _run_fps(xyz_t)

    new_xyz = jnp.transpose(jnp.concatenate([cx, cy, cz], axis=-1), (1, 0, 2))

    xc = jnp.broadcast_to(
        xyz_t.reshape(3, B // 2, 2, N // 128, 128)[:, :, :, None, :, :],
        (3, B // 2, 2, 4, N // 128, 128))
    xc = jnp.transpose(xc, (1, 0, 4, 2, 3, 5)).reshape(B // 2, 3, N // 128, 8, 128)
    nc = jnp.stack([cx[..., 0], cy[..., 0], cz[..., 0]], axis=0)
    nc5 = nc.reshape(3, 4, 128, B // 2, 2)
    qq = jnp.transpose(nc5, (3, 0, 4, 1, 2)).reshape(B // 2, 3, 8, 128)
    basearr = jnp.broadcast_to(
        jnp.arange(B, dtype=jnp.int32).reshape(B // 2, 2, 1) * N,
        (B // 2, 2, 4)).reshape(B // 2, 1, 8, 1)
    knn = _run_knn(xc, qq, basearr)

    knn_flat = (jnp.transpose(knn.reshape(B // 2, K, 2, 4, 128),
                              (0, 2, 3, 4, 1)).reshape(-1))
    fps_flat = (jnp.transpose(fps_idx[..., 0])
                + jnp.arange(B, dtype=jnp.int32)[:, None] * N).reshape(-1)

    table = jnp.concatenate(
        [points.reshape(B * N, D), xyz.reshape(B * N, 3),
         jnp.zeros((B * N, TW - D - 3), jnp.float32)], axis=-1)

    g_knn = _sc_gather(table, knn_flat, TW, chunk=768)
    g_anchor = _sc_gather(table, fps_flat, TW, chunk=B * G // _SC_NW)

    gp, aug = _run_fin(g_knn.reshape(B * G, K, TW),
                       g_anchor.reshape(B * G, 1, TW))

    augmented = aug[..., :10].reshape(B, G, K, 10)
    grouped_points = gp.reshape(B, G, K, 2 * D)
    return new_xyz, augmented, grouped_points
